# baseline (speedup 1.0000x reference)
import functools

import jax
import jax.numpy as jnp
from jax import lax
from jax.experimental import pallas as pl
from jax.experimental.pallas import tpu as pltpu
from jax.experimental.pallas import tpu_sc as plsc

N = 10000
E = 160000
D_IN = 256
H = 16
C = 16
LSTM_H = 32
LH = LSTM_H

NC = 2
NS = 16
NW = NC * NS
CH = 128
NCHUNK = 40
GRP = 10
NG = NCHUNK // GRP
E_PAD = NW * NCHUNK * CH
N_ACC = 10112
RPT = N_ACC // NS

BR = 2000
GRID = N // BR

_sc_mesh = plsc.VectorSubcoreMesh(core_axis_name="c", subcore_axis_name="s",
                                  num_cores=NC, num_subcores=NS)
_SC_PARAMS = pltpu.CompilerParams(use_tc_tiling_on_sc=False)


@functools.partial(
    pl.kernel,
    out_type=jax.ShapeDtypeStruct((NC, 2, N_ACC, H), jnp.float32),
    mesh=_sc_mesh,
    compiler_params=_SC_PARAMS,
    scratch_types=[
        pltpu.VMEM((NCHUNK, CH), jnp.int32),
        pltpu.VMEM((NCHUNK, CH), jnp.int32),
        pltpu.VMEM((CH, H), jnp.float32),
        pltpu.VMEM_SHARED((N_ACC, H), jnp.float32),
        pltpu.VMEM_SHARED((N_ACC, H), jnp.float32),
        pltpu.SemaphoreType.DMA,
    ],
)
def _sc_degrees(src_hbm, dst_hbm, zeros_hbm, ones_hbm, out_hbm,
                sidx, didx, ones_v, acc_o, acc_i, sem):
    c = lax.axis_index("c")
    s = lax.axis_index("s")
    w = s * NC + c
    pltpu.sync_copy(src_hbm.at[w], sidx)
    pltpu.sync_copy(dst_hbm.at[w], didx)
    pltpu.sync_copy(ones_hbm, ones_v)
    r0 = s * RPT
    pltpu.sync_copy(zeros_hbm.at[pl.ds(r0, RPT)], acc_o.at[pl.ds(r0, RPT)])
    pltpu.sync_copy(zeros_hbm.at[pl.ds(r0, RPT)], acc_i.at[pl.ds(r0, RPT)])
    plsc.subcore_barrier()

    def fire(g):
        for b in range(GRP):
            j = g * GRP + b
            pltpu.async_copy(ones_v, acc_o.at[sidx.at[j]], sem, add=True)
            pltpu.async_copy(ones_v, acc_i.at[didx.at[j]], sem, add=True)

    def drain(g):
        for b in range(GRP):
            j = g * GRP + b
            pltpu.make_async_copy(ones_v, acc_o.at[sidx.at[j]], sem).wait()
            pltpu.make_async_copy(ones_v, acc_i.at[didx.at[j]], sem).wait()

    def body(g, carry):
        fire(g)

        @pl.when(g > 0)
        def _():
            drain(g - 1)
        return carry

    lax.fori_loop(0, NG, body, 0)
    drain(NG - 1)
    plsc.subcore_barrier()
    pltpu.sync_copy(acc_o.at[pl.ds(r0, RPT)], out_hbm.at[c, 0, pl.ds(r0, RPT)])
    pltpu.sync_copy(acc_i.at[pl.ds(r0, RPT)], out_hbm.at[c, 1, pl.ds(r0, RPT)])


@functools.partial(
    pl.kernel,
    out_type=jax.ShapeDtypeStruct((NC, N_ACC, H), jnp.float32),
    mesh=_sc_mesh,
    compiler_params=_SC_PARAMS,
    scratch_types=[
        pltpu.VMEM((NCHUNK, CH), jnp.int32),
        pltpu.VMEM((NCHUNK, CH), jnp.int32),
        pltpu.VMEM((3 * GRP, CH, H), jnp.float32),
        pltpu.VMEM_SHARED((N_ACC, H), jnp.float32),
        pltpu.SemaphoreType.DMA,
        pltpu.SemaphoreType.DMA,
    ],
)
def _sc_prop(x_hbm, src_hbm, dst_hbm, zeros_hbm, out_hbm,
             sidx, didx, rows_v, acc, gsem, ssem):
    c = lax.axis_index("c")
    s = lax.axis_index("s")
    w = s * NC + c
    pltpu.sync_copy(src_hbm.at[w], sidx)
    pltpu.sync_copy(dst_hbm.at[w], didx)
    r0 = s * RPT
    pltpu.sync_copy(zeros_hbm.at[pl.ds(r0, RPT)], acc.at[pl.ds(r0, RPT)])
    plsc.subcore_barrier()

    def fire_gathers(g):
        ph = (g % 3) * GRP
        for b in range(GRP):
            pltpu.async_copy(x_hbm.at[sidx.at[g * GRP + b]],
                             rows_v.at[ph + b], gsem)

    def wait_gathers(g):
        ph = (g % 3) * GRP
        for b in range(GRP):
            pltpu.make_async_copy(x_hbm.at[sidx.at[g * GRP + b]],
                                  rows_v.at[ph + b], gsem).wait()

    def fire_scatters(g):
        ph = (g % 3) * GRP
        for b in range(GRP):
            pltpu.async_copy(rows_v.at[ph + b],
                             acc.at[didx.at[g * GRP + b]], ssem, add=True)

    def wait_scatters(g):
        ph = (g % 3) * GRP
        for b in range(GRP):
            pltpu.make_async_copy(rows_v.at[ph + b],
                                  acc.at[didx.at[g * GRP + b]], ssem).wait()

    fire_gathers(0)

    def body(g, carry):
        @pl.when(g + 1 < NG)
        def _():
            fire_gathers(g + 1)
        wait_gathers(g)
        fire_scatters(g)

        @pl.when(g > 0)
        def _():
            wait_scatters(g - 1)
        return carry

    lax.fori_loop(0, NG, body, 0)
    wait_scatters(NG - 1)
    plsc.subcore_barrier()
    pltpu.sync_copy(acc.at[pl.ds(r0, RPT)], out_hbm.at[c, pl.ds(r0, RPT)])


def _blk(shape):
    nlead = len(shape) - 2
    return pl.BlockSpec(shape, lambda i, _n=nlead: (0,) * _n + (i, 0))


def _full(shape):
    return pl.BlockSpec(shape, lambda i: (0,) * len(shape))


_TC_PARAMS = pltpu.CompilerParams(dimension_semantics=("arbitrary",))


def _dotT(x, w):
    return lax.dot_general(x, w, (((1,), (1,)), ((), ())),
                           preferred_element_type=jnp.float32)


def _s1_body(feat_ref, w1_ref, degp_ref, xs1_ref, no_ref, ni_ref):
    y = jnp.dot(feat_ref[...], w1_ref[...], preferred_element_type=jnp.float32)
    dp = degp_ref[...]
    no = lax.rsqrt(dp[0, 0] + dp[1, 0] + 1.0)
    ni = lax.rsqrt(dp[0, 1] + dp[1, 1] + 1.0)
    xs1_ref[...] = y * no
    no_ref[...] = no
    ni_ref[...] = ni


def _tc_s1(feat, W1, degp):
    return pl.pallas_call(
        _s1_body,
        grid=(GRID,),
        in_specs=[_blk((BR, D_IN)), _full((D_IN, H)), _blk((2, 2, BR, H))],
        out_specs=[_blk((BR, H)), _blk((BR, H)), _blk((BR, H))],
        out_shape=[jax.ShapeDtypeStruct((N, H), jnp.float32)] * 3,
        compiler_params=_TC_PARAMS,
    )(feat, W1, degp)


def _s2_body(xs1, ap, ni, no, w2, b1, h1_ref, xs2_ref):
    a = ap[...]
    agg = a[0] + a[1] + xs1[...]
    h1 = jnp.maximum(agg * ni[...] + b1[...][None, :], 0.0)
    h1_ref[...] = h1
    xs2_ref[...] = jnp.dot(h1 * no[...], w2[...],
                           preferred_element_type=jnp.float32)


def _tc_s2(xs1, ap, ni, no, W2, b1):
    return pl.pallas_call(
        _s2_body,
        grid=(GRID,),
        in_specs=[_blk((BR, H)), _blk((2, BR, H)), _blk((BR, H)),
                  _blk((BR, H)), _full((H, H)), _full((H,))],
        out_specs=[_blk((BR, H)), _blk((BR, H))],
        out_shape=[jax.ShapeDtypeStruct((N, H), jnp.float32)] * 2,
        compiler_params=_TC_PARAMS,
    )(xs1, ap, ni, no, W2, b1)


def _gates(g, lo):
    i = jax.nn.sigmoid(g[:, lo + 0 * LH:lo + 1 * LH])
    f = jax.nn.sigmoid(g[:, lo + 1 * LH:lo + 2 * LH])
    gg = jnp.tanh(g[:, lo + 2 * LH:lo + 3 * LH])
    o = jax.nn.sigmoid(g[:, lo + 3 * LH:lo + 4 * LH])
    return i, f, gg, o


def _s3_body(h1, xs2, ap, ni, no, b2, wcat, whbd, bcat, awbd, attb, hs_ref):
    x0 = h1[...]
    a = ap[...]
    agg = a[0] + a[1] + xs2[...]
    x1 = jnp.maximum(agg * ni[...] + b2[...][None, :], 0.0)
    bc = bcat[...][None, :]
    g0 = _dotT(x0, wcat[...]) + bc
    g1 = _dotT(x1, wcat[...]) + bc
    i, f, gg, o = _gates(g0, 0)
    cf0 = i * gg
    hf0 = o * jnp.tanh(cf0)
    i, f, gg, o = _gates(g1, 4 * LH)
    cr1 = i * gg
    hr1 = o * jnp.tanh(cr1)
    hcat = jnp.concatenate([hf0, hr1], axis=-1)
    gh = _dotT(hcat, whbd[...])
    i, f, gg, o = _gates(g1 + gh, 0)
    cf1 = f * cf0 + i * gg
    hf1 = o * jnp.tanh(cf1)
    i, f, gg, o = _gates(g0 + gh, 4 * LH)
    cr0 = f * cr1 + i * gg
    hr0 = o * jnp.tanh(cr0)
    hall = jnp.concatenate([hf0, hr0, hf1, hr1], axis=-1)
    s = (jnp.dot(hall, awbd[...], preferred_element_type=jnp.float32)
         + attb[...][None, :])
    s0 = s[:, 0:1]
    s1 = s[:, 1:2]
    m = jnp.maximum(s0, s1)
    e0 = jnp.exp(s0 - m)
    e1 = jnp.exp(s1 - m)
    inv = 1.0 / (e0 + e1)
    h = (e0 * inv) * x0 + (e1 * inv) * x1
    hs_ref[...] = h * no[...]


def _tc_s3(h1, xs2, ap, ni, no, b2, wcat, whbd, bcat, awbd, att_b):
    return pl.pallas_call(
        _s3_body,
        grid=(GRID,),
        in_specs=[_blk((BR, H)), _blk((BR, H)), _blk((2, BR, H)),
                  _blk((BR, H)), _blk((BR, H)), _full((H,)),
                  _full((8 * LH, H)), _full((8 * LH, 2 * LH)),
                  _full((8 * LH,)), _full((4 * LH, 2)), _full((1,))],
        out_specs=[_blk((BR, H))],
        out_shape=[jax.ShapeDtypeStruct((N, H), jnp.float32)],
        compiler_params=_TC_PARAMS,
    )(h1, xs2, ap, ni, no, b2, wcat, whbd, bcat, awbd, att_b)[0]


def _s4_body(hs, ap, ni, linw, lin_b, out_ref):
    a = ap[...]
    t = (a[0] + a[1] + hs[...]) * ni[...]
    z = _dotT(t, linw[...]) + lin_b[...][None, :]
    m = jnp.max(z, axis=-1, keepdims=True)
    zz = z - m
    out_ref[...] = zz - jnp.log(jnp.sum(jnp.exp(zz), axis=-1, keepdims=True))


def _tc_s4(hs, ap, ni, lin_W, lin_b):
    return pl.pallas_call(
        _s4_body,
        grid=(GRID,),
        in_specs=[_blk((BR, H)), _blk((2, BR, H)), _blk((BR, H)),
                  _full((C, H)), _full((C,))],
        out_specs=[_blk((BR, C))],
        out_shape=[jax.ShapeDtypeStruct((N, C), jnp.float32)],
        compiler_params=_TC_PARAMS,
    )(hs, ap, ni, lin_W, lin_b)[0]


def kernel(feat, W1, b1, W2, b2, W_ih_f, W_hh_f, b_ih_f, b_hh_f,
           W_ih_r, W_hh_r, b_ih_r, b_hh_r, att_W, att_b, lin_W, lin_b,
           edge_index):
    src = edge_index[0].astype(jnp.int32)
    dst = edge_index[1].astype(jnp.int32)
    pad = E_PAD - E
    src_p = jnp.concatenate(
        [src, jnp.zeros((pad,), jnp.int32)]).reshape(NW, NCHUNK, CH)
    dst_p = jnp.concatenate(
        [dst, jnp.full((pad,), N, jnp.int32)]).reshape(NW, NCHUNK, CH)
    zeros = jnp.zeros((N_ACC, H), jnp.float32)
    ones = jnp.ones((CH, H), jnp.float32)

    degp = _sc_degrees(src_p, dst_p, zeros, ones)
    xs1, no16, ni16 = _tc_s1(feat, W1, degp)

    a = _sc_prop(xs1, src_p, dst_p, zeros)
    h1, xs2 = _tc_s2(xs1, a, ni16, no16, W2, b1)

    a = _sc_prop(xs2, src_p, dst_p, zeros)
    wcat = jnp.concatenate([W_ih_f, W_ih_r], axis=0)
    zb = jnp.zeros((4 * LH, LH), jnp.float32)
    whbd = jnp.concatenate(
        [jnp.concatenate([W_hh_f, zb], axis=1),
         jnp.concatenate([zb, W_hh_r], axis=1)], axis=0)
    bcat = jnp.concatenate([b_ih_f + b_hh_f, b_ih_r + b_hh_r])
    aw = att_W[:, 0]
    za = jnp.zeros((2 * LH,), jnp.float32)
    awbd = jnp.stack([jnp.concatenate([aw, za]),
                      jnp.concatenate([za, aw])], axis=1)
    hs = _tc_s3(h1, xs2, a, ni16, no16, b2, wcat, whbd, bcat, awbd, att_b)

    a = _sc_prop(hs, src_p, dst_p, zeros)
    return _tc_s4(hs, a, ni16, lin_W, lin_b)


# speedup vs baseline: 1.2396x; 1.2396x over previous
import functools

import jax
import jax.numpy as jnp
from jax import lax
from jax.experimental import pallas as pl
from jax.experimental.pallas import tpu as pltpu
from jax.experimental.pallas import tpu_sc as plsc

N = 10000
E = 160000
D_IN = 256
H = 16
C = 16
LSTM_H = 32
LH = LSTM_H

NC = 2
NS = 16
NW = NC * NS
CH = 125
NCHUNK = 40
GRP = 8
NG = NCHUNK // GRP
N_ACC = 10112
RPT = N_ACC // NS

BR = 2000
GRID = N // BR

_sc_mesh = plsc.VectorSubcoreMesh(core_axis_name="c", subcore_axis_name="s",
                                  num_cores=NC, num_subcores=NS)
_SC_PARAMS = pltpu.CompilerParams(use_tc_tiling_on_sc=False)


@functools.partial(
    pl.kernel,
    out_type=jax.ShapeDtypeStruct((NC, 2, N_ACC, H), jnp.float32),
    mesh=_sc_mesh,
    compiler_params=_SC_PARAMS,
    scratch_types=[
        pltpu.VMEM((NCHUNK, CH), jnp.int32),
        pltpu.VMEM((NCHUNK, CH), jnp.int32),
        pltpu.VMEM((CH, H), jnp.float32),
        pltpu.VMEM_SHARED((N_ACC, H), jnp.float32),
        pltpu.VMEM_SHARED((N_ACC, H), jnp.float32),
        pltpu.SemaphoreType.DMA,
    ],
)
def _sc_degrees(src_hbm, dst_hbm, zeros_hbm, ones_hbm, out_hbm,
                sidx, didx, ones_v, acc_o, acc_i, sem):
    c = lax.axis_index("c")
    s = lax.axis_index("s")
    w = s * NC + c
    pltpu.sync_copy(src_hbm.at[w], sidx)
    pltpu.sync_copy(dst_hbm.at[w], didx)
    pltpu.sync_copy(ones_hbm, ones_v)
    r0 = s * RPT
    pltpu.sync_copy(zeros_hbm.at[pl.ds(r0, RPT)], acc_o.at[pl.ds(r0, RPT)])
    pltpu.sync_copy(zeros_hbm.at[pl.ds(r0, RPT)], acc_i.at[pl.ds(r0, RPT)])
    plsc.subcore_barrier()

    def group(g, carry):
        descs = []
        for b in range(GRP):
            j = g * GRP + b
            descs.append(pltpu.async_copy(ones_v, acc_o.at[sidx.at[j]],
                                          sem, add=True))
            descs.append(pltpu.async_copy(ones_v, acc_i.at[didx.at[j]],
                                          sem, add=True))
        for d in descs:
            d.wait()
        return carry

    lax.fori_loop(0, NG, group, 0)
    plsc.subcore_barrier()
    pltpu.sync_copy(acc_o.at[pl.ds(r0, RPT)], out_hbm.at[c, 0, pl.ds(r0, RPT)])
    pltpu.sync_copy(acc_i.at[pl.ds(r0, RPT)], out_hbm.at[c, 1, pl.ds(r0, RPT)])


@functools.partial(
    pl.kernel,
    out_type=jax.ShapeDtypeStruct((NC, N_ACC, H), jnp.float32),
    mesh=_sc_mesh,
    compiler_params=_SC_PARAMS,
    scratch_types=[
        pltpu.VMEM((NCHUNK, CH), jnp.int32),
        pltpu.VMEM((NCHUNK, CH), jnp.int32),
        pltpu.VMEM((2 * GRP, CH, H), jnp.float32),
        pltpu.VMEM_SHARED((N_ACC, H), jnp.float32),
        pltpu.SemaphoreType.DMA,
        pltpu.SemaphoreType.DMA,
    ],
)
def _sc_prop(x_hbm, src_hbm, dst_hbm, zeros_hbm, out_hbm,
             sidx, didx, rows_v, acc, gsem, ssem):
    c = lax.axis_index("c")
    s = lax.axis_index("s")
    w = s * NC + c
    pltpu.sync_copy(src_hbm.at[w], sidx)
    pltpu.sync_copy(dst_hbm.at[w], didx)
    r0 = s * RPT
    pltpu.sync_copy(zeros_hbm.at[pl.ds(r0, RPT)], acc.at[pl.ds(r0, RPT)])
    plsc.subcore_barrier()

    def fire_gathers(g):
        half = (g % 2) * GRP
        for b in range(GRP):
            pltpu.async_copy(x_hbm.at[sidx.at[g * GRP + b]],
                             rows_v.at[half + b], gsem)

    def wait_gathers(g):
        half = (g % 2) * GRP
        for b in range(GRP):
            pltpu.make_async_copy(x_hbm.at[sidx.at[g * GRP + b]],
                                  rows_v.at[half + b], gsem).wait()

    def fire_scatters(g):
        half = (g % 2) * GRP
        for b in range(GRP):
            pltpu.async_copy(rows_v.at[half + b],
                             acc.at[didx.at[g * GRP + b]], ssem, add=True)

    def wait_scatters(g):
        half = (g % 2) * GRP
        for b in range(GRP):
            pltpu.make_async_copy(rows_v.at[half + b],
                                  acc.at[didx.at[g * GRP + b]], ssem).wait()

    fire_gathers(0)

    def body(g, carry):
        @pl.when(g + 1 < NG)
        def _():
            fire_gathers(g + 1)
        wait_gathers(g)
        fire_scatters(g)
        wait_scatters(g)
        return carry

    lax.fori_loop(0, NG, body, 0)
    plsc.subcore_barrier()
    pltpu.sync_copy(acc.at[pl.ds(r0, RPT)], out_hbm.at[c, pl.ds(r0, RPT)])


def _blk(shape):
    nlead = len(shape) - 2
    return pl.BlockSpec(shape, lambda i, _n=nlead: (0,) * _n + (i, 0))


def _full(shape):
    return pl.BlockSpec(shape, lambda i: (0,) * len(shape))


_TC_PARAMS = pltpu.CompilerParams(dimension_semantics=("arbitrary",))


def _dotT(x, w):
    return lax.dot_general(x, w, (((1,), (1,)), ((), ())),
                           preferred_element_type=jnp.float32)


def _s1_body(feat_ref, w1_ref, degp_ref, xs1_ref, no_ref, ni_ref):
    y = jnp.dot(feat_ref[...], w1_ref[...], preferred_element_type=jnp.float32)
    dp = degp_ref[...]
    no = lax.rsqrt(dp[0, 0] + dp[1, 0] + 1.0)
    ni = lax.rsqrt(dp[0, 1] + dp[1, 1] + 1.0)
    xs1_ref[...] = y * no
    no_ref[...] = no
    ni_ref[...] = ni


def _tc_s1(feat, W1, degp):
    return pl.pallas_call(
        _s1_body,
        grid=(GRID,),
        in_specs=[_blk((BR, D_IN)), _full((D_IN, H)), _blk((2, 2, BR, H))],
        out_specs=[_blk((BR, H)), _blk((BR, H)), _blk((BR, H))],
        out_shape=[jax.ShapeDtypeStruct((N, H), jnp.float32)] * 3,
        compiler_params=_TC_PARAMS,
    )(feat, W1, degp)


def _s2_body(xs1, ap, ni, no, w2, b1, h1_ref, xs2_ref):
    a = ap[...]
    agg = a[0] + a[1] + xs1[...]
    h1 = jnp.maximum(agg * ni[...] + b1[...][None, :], 0.0)
    h1_ref[...] = h1
    xs2_ref[...] = jnp.dot(h1 * no[...], w2[...],
                           preferred_element_type=jnp.float32)


def _tc_s2(xs1, ap, ni, no, W2, b1):
    return pl.pallas_call(
        _s2_body,
        grid=(GRID,),
        in_specs=[_blk((BR, H)), _blk((2, BR, H)), _blk((BR, H)),
                  _blk((BR, H)), _full((H, H)), _full((H,))],
        out_specs=[_blk((BR, H)), _blk((BR, H))],
        out_shape=[jax.ShapeDtypeStruct((N, H), jnp.float32)] * 2,
        compiler_params=_TC_PARAMS,
    )(xs1, ap, ni, no, W2, b1)


def _gates(g, lo):
    i = jax.nn.sigmoid(g[:, lo + 0 * LH:lo + 1 * LH])
    f = jax.nn.sigmoid(g[:, lo + 1 * LH:lo + 2 * LH])
    gg = jnp.tanh(g[:, lo + 2 * LH:lo + 3 * LH])
    o = jax.nn.sigmoid(g[:, lo + 3 * LH:lo + 4 * LH])
    return i, f, gg, o


def _s3_body(h1, xs2, ap, ni, no, b2, wcat, whbd, bcat, awbd, attb, hs_ref):
    x0 = h1[...]
    a = ap[...]
    agg = a[0] + a[1] + xs2[...]
    x1 = jnp.maximum(agg * ni[...] + b2[...][None, :], 0.0)
    bc = bcat[...][None, :]
    g0 = _dotT(x0, wcat[...]) + bc
    g1 = _dotT(x1, wcat[...]) + bc
    i, f, gg, o = _gates(g0, 0)
    cf0 = i * gg
    hf0 = o * jnp.tanh(cf0)
    i, f, gg, o = _gates(g1, 4 * LH)
    cr1 = i * gg
    hr1 = o * jnp.tanh(cr1)
    hcat = jnp.concatenate([hf0, hr1], axis=-1)
    gh = _dotT(hcat, whbd[...])
    i, f, gg, o = _gates(g1 + gh, 0)
    cf1 = f * cf0 + i * gg
    hf1 = o * jnp.tanh(cf1)
    i, f, gg, o = _gates(g0 + gh, 4 * LH)
    cr0 = f * cr1 + i * gg
    hr0 = o * jnp.tanh(cr0)
    hall = jnp.concatenate([hf0, hr0, hf1, hr1], axis=-1)
    s = (jnp.dot(hall, awbd[...], preferred_element_type=jnp.float32)
         + attb[...][None, :])
    s0 = s[:, 0:1]
    s1 = s[:, 1:2]
    m = jnp.maximum(s0, s1)
    e0 = jnp.exp(s0 - m)
    e1 = jnp.exp(s1 - m)
    inv = 1.0 / (e0 + e1)
    h = (e0 * inv) * x0 + (e1 * inv) * x1
    hs_ref[...] = h * no[...]


def _tc_s3(h1, xs2, ap, ni, no, b2, wcat, whbd, bcat, awbd, att_b):
    return pl.pallas_call(
        _s3_body,
        grid=(GRID,),
        in_specs=[_blk((BR, H)), _blk((BR, H)), _blk((2, BR, H)),
                  _blk((BR, H)), _blk((BR, H)), _full((H,)),
                  _full((8 * LH, H)), _full((8 * LH, 2 * LH)),
                  _full((8 * LH,)), _full((4 * LH, 2)), _full((1,))],
        out_specs=[_blk((BR, H))],
        out_shape=[jax.ShapeDtypeStruct((N, H), jnp.float32)],
        compiler_params=_TC_PARAMS,
    )(h1, xs2, ap, ni, no, b2, wcat, whbd, bcat, awbd, att_b)[0]


def _s4_body(hs, ap, ni, linw, lin_b, out_ref):
    a = ap[...]
    t = (a[0] + a[1] + hs[...]) * ni[...]
    z = _dotT(t, linw[...]) + lin_b[...][None, :]
    m = jnp.max(z, axis=-1, keepdims=True)
    zz = z - m
    out_ref[...] = zz - jnp.log(jnp.sum(jnp.exp(zz), axis=-1, keepdims=True))


def _tc_s4(hs, ap, ni, lin_W, lin_b):
    return pl.pallas_call(
        _s4_body,
        grid=(GRID,),
        in_specs=[_blk((BR, H)), _blk((2, BR, H)), _blk((BR, H)),
                  _full((C, H)), _full((C,))],
        out_specs=[_blk((BR, C))],
        out_shape=[jax.ShapeDtypeStruct((N, C), jnp.float32)],
        compiler_params=_TC_PARAMS,
    )(hs, ap, ni, lin_W, lin_b)[0]


def kernel(feat, W1, b1, W2, b2, W_ih_f, W_hh_f, b_ih_f, b_hh_f,
           W_ih_r, W_hh_r, b_ih_r, b_hh_r, att_W, att_b, lin_W, lin_b,
           edge_index):
    src_p = edge_index[0].astype(jnp.int32).reshape(NW, NCHUNK, CH)
    dst_p = edge_index[1].astype(jnp.int32).reshape(NW, NCHUNK, CH)
    zeros = jnp.zeros((N_ACC, H), jnp.float32)
    ones = jnp.ones((CH, H), jnp.float32)

    degp = _sc_degrees(src_p, dst_p, zeros, ones)
    xs1, no16, ni16 = _tc_s1(feat, W1, degp)

    a = _sc_prop(xs1, src_p, dst_p, zeros)
    h1, xs2 = _tc_s2(xs1, a, ni16, no16, W2, b1)

    a = _sc_prop(xs2, src_p, dst_p, zeros)
    wcat = jnp.concatenate([W_ih_f, W_ih_r], axis=0)
    zb = jnp.zeros((4 * LH, LH), jnp.float32)
    whbd = jnp.concatenate(
        [jnp.concatenate([W_hh_f, zb], axis=1),
         jnp.concatenate([zb, W_hh_r], axis=1)], axis=0)
    bcat = jnp.concatenate([b_ih_f + b_hh_f, b_ih_r + b_hh_r])
    aw = att_W[:, 0]
    za = jnp.zeros((2 * LH,), jnp.float32)
    awbd = jnp.stack([jnp.concatenate([aw, za]),
                      jnp.concatenate([za, aw])], axis=1)
    hs = _tc_s3(h1, xs2, a, ni16, no16, b2, wcat, whbd, bcat, awbd, att_b)

    a = _sc_prop(hs, src_p, dst_p, zeros)
    return _tc_s4(hs, a, ni16, lin_W, lin_b)


# speedup vs baseline: 1.2445x; 1.0040x over previous
import functools

import jax
import jax.numpy as jnp
from jax import lax
from jax.experimental import pallas as pl
from jax.experimental.pallas import tpu as pltpu
from jax.experimental.pallas import tpu_sc as plsc

N = 10000
E = 160000
D_IN = 256
H = 16
C = 16
LSTM_H = 32
LH = LSTM_H

NC = 2
NS = 16
NW = NC * NS
CH = 125
NCHUNK = 40
GRP = 10
NG = NCHUNK // GRP
N_ACC = 10112
RPT = N_ACC // NS

BR = 2000
GRID = N // BR

_sc_mesh = plsc.VectorSubcoreMesh(core_axis_name="c", subcore_axis_name="s",
                                  num_cores=NC, num_subcores=NS)
_SC_PARAMS = pltpu.CompilerParams(use_tc_tiling_on_sc=False)


@functools.partial(
    pl.kernel,
    out_type=jax.ShapeDtypeStruct((NC, 2, N_ACC, H), jnp.float32),
    mesh=_sc_mesh,
    compiler_params=_SC_PARAMS,
    scratch_types=[
        pltpu.VMEM((NCHUNK, CH), jnp.int32),
        pltpu.VMEM((NCHUNK, CH), jnp.int32),
        pltpu.VMEM((CH, H), jnp.float32),
        pltpu.VMEM_SHARED((N_ACC, H), jnp.float32),
        pltpu.VMEM_SHARED((N_ACC, H), jnp.float32),
        pltpu.SemaphoreType.DMA,
    ],
)
def _sc_degrees(src_hbm, dst_hbm, zeros_hbm, ones_hbm, out_hbm,
                sidx, didx, ones_v, acc_o, acc_i, sem):
    c = lax.axis_index("c")
    s = lax.axis_index("s")
    w = s * NC + c
    pltpu.sync_copy(src_hbm.at[w], sidx)
    pltpu.sync_copy(dst_hbm.at[w], didx)
    pltpu.sync_copy(ones_hbm, ones_v)
    r0 = s * RPT
    pltpu.sync_copy(zeros_hbm.at[pl.ds(r0, RPT)], acc_o.at[pl.ds(r0, RPT)])
    pltpu.sync_copy(zeros_hbm.at[pl.ds(r0, RPT)], acc_i.at[pl.ds(r0, RPT)])
    plsc.subcore_barrier()

    def group(g, carry):
        descs = []
        for b in range(GRP):
            j = g * GRP + b
            descs.append(pltpu.async_copy(ones_v, acc_o.at[sidx.at[j]],
                                          sem, add=True))
            descs.append(pltpu.async_copy(ones_v, acc_i.at[didx.at[j]],
                                          sem, add=True))
        for d in descs:
            d.wait()
        return carry

    lax.fori_loop(0, NG, group, 0)
    plsc.subcore_barrier()
    pltpu.sync_copy(acc_o.at[pl.ds(r0, RPT)], out_hbm.at[c, 0, pl.ds(r0, RPT)])
    pltpu.sync_copy(acc_i.at[pl.ds(r0, RPT)], out_hbm.at[c, 1, pl.ds(r0, RPT)])


@functools.partial(
    pl.kernel,
    out_type=jax.ShapeDtypeStruct((NC, N_ACC, H), jnp.float32),
    mesh=_sc_mesh,
    compiler_params=_SC_PARAMS,
    scratch_types=[
        pltpu.VMEM((NCHUNK, CH), jnp.int32),
        pltpu.VMEM((NCHUNK, CH), jnp.int32),
        pltpu.VMEM((2 * GRP, CH, H), jnp.float32),
        pltpu.VMEM_SHARED((N_ACC, H), jnp.float32),
        pltpu.SemaphoreType.DMA,
        pltpu.SemaphoreType.DMA,
    ],
)
def _sc_prop(x_hbm, src_hbm, dst_hbm, zeros_hbm, out_hbm,
             sidx, didx, rows_v, acc, gsem, ssem):
    c = lax.axis_index("c")
    s = lax.axis_index("s")
    w = s * NC + c
    pltpu.sync_copy(src_hbm.at[w], sidx)
    pltpu.sync_copy(dst_hbm.at[w], didx)
    r0 = s * RPT
    pltpu.sync_copy(zeros_hbm.at[pl.ds(r0, RPT)], acc.at[pl.ds(r0, RPT)])
    plsc.subcore_barrier()

    def fire_gathers(g):
        half = (g % 2) * GRP
        for b in range(GRP):
            pltpu.async_copy(x_hbm.at[sidx.at[g * GRP + b]],
                             rows_v.at[half + b], gsem)

    def wait_gathers(g):
        half = (g % 2) * GRP
        for b in range(GRP):
            pltpu.make_async_copy(x_hbm.at[sidx.at[g * GRP + b]],
                                  rows_v.at[half + b], gsem).wait()

    def fire_scatters(g):
        half = (g % 2) * GRP
        for b in range(GRP):
            pltpu.async_copy(rows_v.at[half + b],
                             acc.at[didx.at[g * GRP + b]], ssem, add=True)

    def wait_scatters(g):
        half = (g % 2) * GRP
        for b in range(GRP):
            pltpu.make_async_copy(rows_v.at[half + b],
                                  acc.at[didx.at[g * GRP + b]], ssem).wait()

    fire_gathers(0)

    def body(g, carry):
        @pl.when(g + 1 < NG)
        def _():
            fire_gathers(g + 1)
        wait_gathers(g)
        fire_scatters(g)
        wait_scatters(g)
        return carry

    lax.fori_loop(0, NG, body, 0)
    plsc.subcore_barrier()
    pltpu.sync_copy(acc.at[pl.ds(r0, RPT)], out_hbm.at[c, pl.ds(r0, RPT)])


def _blk(shape):
    nlead = len(shape) - 2
    return pl.BlockSpec(shape, lambda i, _n=nlead: (0,) * _n + (i, 0))


def _full(shape):
    return pl.BlockSpec(shape, lambda i: (0,) * len(shape))


_TC_PARAMS = pltpu.CompilerParams(dimension_semantics=("arbitrary",))


def _dotT(x, w):
    return lax.dot_general(x, w, (((1,), (1,)), ((), ())),
                           preferred_element_type=jnp.float32)


def _s1_body(feat_ref, w1_ref, degp_ref, xs1_ref, no_ref, ni_ref):
    y = jnp.dot(feat_ref[...], w1_ref[...], preferred_element_type=jnp.float32)
    dp = degp_ref[...]
    no = lax.rsqrt(dp[0, 0] + dp[1, 0] + 1.0)
    ni = lax.rsqrt(dp[0, 1] + dp[1, 1] + 1.0)
    xs1_ref[...] = y * no
    no_ref[...] = no
    ni_ref[...] = ni


def _tc_s1(feat, W1, degp):
    return pl.pallas_call(
        _s1_body,
        grid=(GRID,),
        in_specs=[_blk((BR, D_IN)), _full((D_IN, H)), _blk((2, 2, BR, H))],
        out_specs=[_blk((BR, H)), _blk((BR, H)), _blk((BR, H))],
        out_shape=[jax.ShapeDtypeStruct((N, H), jnp.float32)] * 3,
        compiler_params=_TC_PARAMS,
    )(feat, W1, degp)


def _s2_body(xs1, ap, ni, no, w2, b1, h1_ref, xs2_ref):
    a = ap[...]
    agg = a[0] + a[1] + xs1[...]
    h1 = jnp.maximum(agg * ni[...] + b1[...][None, :], 0.0)
    h1_ref[...] = h1
    xs2_ref[...] = jnp.dot(h1 * no[...], w2[...],
                           preferred_element_type=jnp.float32)


def _tc_s2(xs1, ap, ni, no, W2, b1):
    return pl.pallas_call(
        _s2_body,
        grid=(GRID,),
        in_specs=[_blk((BR, H)), _blk((2, BR, H)), _blk((BR, H)),
                  _blk((BR, H)), _full((H, H)), _full((H,))],
        out_specs=[_blk((BR, H)), _blk((BR, H))],
        out_shape=[jax.ShapeDtypeStruct((N, H), jnp.float32)] * 2,
        compiler_params=_TC_PARAMS,
    )(xs1, ap, ni, no, W2, b1)


def _gates(g, lo):
    i = jax.nn.sigmoid(g[:, lo + 0 * LH:lo + 1 * LH])
    f = jax.nn.sigmoid(g[:, lo + 1 * LH:lo + 2 * LH])
    gg = jnp.tanh(g[:, lo + 2 * LH:lo + 3 * LH])
    o = jax.nn.sigmoid(g[:, lo + 3 * LH:lo + 4 * LH])
    return i, f, gg, o


def _s3_body(h1, xs2, ap, ni, no, b2, wcat, whbd, bcat, awbd, attb, hs_ref):
    x0 = h1[...]
    a = ap[...]
    agg = a[0] + a[1] + xs2[...]
    x1 = jnp.maximum(agg * ni[...] + b2[...][None, :], 0.0)
    bc = bcat[...][None, :]
    g0 = _dotT(x0, wcat[...]) + bc
    g1 = _dotT(x1, wcat[...]) + bc
    i, f, gg, o = _gates(g0, 0)
    cf0 = i * gg
    hf0 = o * jnp.tanh(cf0)
    i, f, gg, o = _gates(g1, 4 * LH)
    cr1 = i * gg
    hr1 = o * jnp.tanh(cr1)
    hcat = jnp.concatenate([hf0, hr1], axis=-1)
    gh = _dotT(hcat, whbd[...])
    i, f, gg, o = _gates(g1 + gh, 0)
    cf1 = f * cf0 + i * gg
    hf1 = o * jnp.tanh(cf1)
    i, f, gg, o = _gates(g0 + gh, 4 * LH)
    cr0 = f * cr1 + i * gg
    hr0 = o * jnp.tanh(cr0)
    hall = jnp.concatenate([hf0, hr0, hf1, hr1], axis=-1)
    s = (jnp.dot(hall, awbd[...], preferred_element_type=jnp.float32)
         + attb[...][None, :])
    s0 = s[:, 0:1]
    s1 = s[:, 1:2]
    m = jnp.maximum(s0, s1)
    e0 = jnp.exp(s0 - m)
    e1 = jnp.exp(s1 - m)
    inv = 1.0 / (e0 + e1)
    h = (e0 * inv) * x0 + (e1 * inv) * x1
    hs_ref[...] = h * no[...]


def _tc_s3(h1, xs2, ap, ni, no, b2, wcat, whbd, bcat, awbd, att_b):
    return pl.pallas_call(
        _s3_body,
        grid=(GRID,),
        in_specs=[_blk((BR, H)), _blk((BR, H)), _blk((2, BR, H)),
                  _blk((BR, H)), _blk((BR, H)), _full((H,)),
                  _full((8 * LH, H)), _full((8 * LH, 2 * LH)),
                  _full((8 * LH,)), _full((4 * LH, 2)), _full((1,))],
        out_specs=[_blk((BR, H))],
        out_shape=[jax.ShapeDtypeStruct((N, H), jnp.float32)],
        compiler_params=_TC_PARAMS,
    )(h1, xs2, ap, ni, no, b2, wcat, whbd, bcat, awbd, att_b)[0]


def _s4_body(hs, ap, ni, linw, lin_b, out_ref):
    a = ap[...]
    t = (a[0] + a[1] + hs[...]) * ni[...]
    z = _dotT(t, linw[...]) + lin_b[...][None, :]
    m = jnp.max(z, axis=-1, keepdims=True)
    zz = z - m
    out_ref[...] = zz - jnp.log(jnp.sum(jnp.exp(zz), axis=-1, keepdims=True))


def _tc_s4(hs, ap, ni, lin_W, lin_b):
    return pl.pallas_call(
        _s4_body,
        grid=(GRID,),
        in_specs=[_blk((BR, H)), _blk((2, BR, H)), _blk((BR, H)),
                  _full((C, H)), _full((C,))],
        out_specs=[_blk((BR, C))],
        out_shape=[jax.ShapeDtypeStruct((N, C), jnp.float32)],
        compiler_params=_TC_PARAMS,
    )(hs, ap, ni, lin_W, lin_b)[0]


def kernel(feat, W1, b1, W2, b2, W_ih_f, W_hh_f, b_ih_f, b_hh_f,
           W_ih_r, W_hh_r, b_ih_r, b_hh_r, att_W, att_b, lin_W, lin_b,
           edge_index):
    src_p = edge_index[0].astype(jnp.int32).reshape(NW, NCHUNK, CH)
    dst_p = edge_index[1].astype(jnp.int32).reshape(NW, NCHUNK, CH)
    zeros = jnp.zeros((N_ACC, H), jnp.float32)
    ones = jnp.ones((CH, H), jnp.float32)

    degp = _sc_degrees(src_p, dst_p, zeros, ones)
    xs1, no16, ni16 = _tc_s1(feat, W1, degp)

    a = _sc_prop(xs1, src_p, dst_p, zeros)
    h1, xs2 = _tc_s2(xs1, a, ni16, no16, W2, b1)

    a = _sc_prop(xs2, src_p, dst_p, zeros)
    wcat = jnp.concatenate([W_ih_f, W_ih_r], axis=0)
    zb = jnp.zeros((4 * LH, LH), jnp.float32)
    whbd = jnp.concatenate(
        [jnp.concatenate([W_hh_f, zb], axis=1),
         jnp.concatenate([zb, W_hh_r], axis=1)], axis=0)
    bcat = jnp.concatenate([b_ih_f + b_hh_f, b_ih_r + b_hh_r])
    aw = att_W[:, 0]
    za = jnp.zeros((2 * LH,), jnp.float32)
    awbd = jnp.stack([jnp.concatenate([aw, za]),
                      jnp.concatenate([za, aw])], axis=1)
    hs = _tc_s3(h1, xs2, a, ni16, no16, b2, wcat, whbd, bcat, awbd, att_b)

    a = _sc_prop(hs, src_p, dst_p, zeros)
    return _tc_s4(hs, a, ni16, lin_W, lin_b)
